# baseline (speedup 1.0000x reference)
import functools

import jax
import jax.numpy as jnp
from jax.experimental import pallas as pl
from jax.experimental.pallas import tpu as pltpu

NGF = 64
N_BLOCKS = 16
DEF_GROUPS = 8
T_REFS = 2
LANES = 128


def _conv2d(x, w, b):
    out = jax.lax.conv_general_dilated(x, w, (1, 1), 'SAME', dimension_numbers=('NCHW', 'OIHW', 'NCHW'))
    return out + b[None, :, None, None]


def _lrelu(x):
    return jnp.where(x >= 0, x, 0.1 * x)


def _prelu(x, a):
    return jnp.where(x >= 0, x, a[None, :, None, None] * x)


def _resblock(x, p):
    return x + _conv2d(jax.nn.relu(_conv2d(x, p['w1'], p['b1'])), p['w2'], p['b2'])


def _run_body(x, plist):
    for p in plist:
        x = _resblock(x, p)
    return x


def _pixel_shuffle(x, r):
    b, c, h, w = x.shape
    x = x.reshape(b, c // (r * r), r, r, h, w)
    return x.transpose(0, 1, 4, 2, 5, 3).reshape(b, c // (r * r), h * r, w * r)


def _spatial_pad(f):
    h, w = f.shape[-2:]
    ph, pw = (4 - h % 4) % 4, (4 - w % 4) % 4
    if ph or pw:
        f = jnp.pad(f, ((0, 0), (0, 0), (0, ph), (0, pw)), mode='reflect')
    return f


def _gather_kernel(H, W, G, Cg, bnd_ref, yx_ref, xg_ref, offs_ref, m_ref, o_ref):
    K = 9
    yf = yx_ref[0, pl.ds(0, 1), :]
    xf = yx_ref[0, pl.ds(1, 1), :]
    for g in range(G):
        for k in range(K):
            ch = g * K + k
            kyf = float(k // 3 - 1)
            kxf = float(k % 3 - 1)
            dy = offs_ref[0, 0, pl.ds(2 * ch, 1), :]
            dx = offs_ref[0, 0, pl.ds(2 * ch + 1, 1), :]
            mm = m_ref[0, 0, pl.ds(ch, 1), :]
            py = (yf + kyf) + dy
            px = (xf + kxf) + dx
            y0 = jnp.floor(py)
            x0 = jnp.floor(px)
            ty = py - y0
            tx = px - x0
            y1 = y0 + 1.0
            x1 = x0 + 1.0

            def cdata(yi, xi):
                valid = ((yi >= 0) & (yi <= H - 1) & (xi >= 0) & (xi <= W - 1)).astype(jnp.float32)
                yc = jnp.clip(yi, 0, H - 1).astype(jnp.int32)
                xc = jnp.clip(xi, 0, W - 1).astype(jnp.int32)
                idx = yc * W + xc
                lidx = jnp.broadcast_to(idx & (LANES - 1), (Cg, LANES))
                hi = idx >> 7
                return valid, lidx, hi

            v00, l00, h00 = cdata(y0, x0)
            v01, l01, h01 = cdata(y0, x1)
            v10, l10, h10 = cdata(y1, x0)
            v11, l11, h11 = cdata(y1, x1)

            clo = bnd_ref[0, 0, 0, 2 * ch]
            chi = bnd_ref[0, 0, 0, 2 * ch + 1]

            def body(c, carry):
                a00, a01, a10, a11 = carry
                chunk = xg_ref[0, g, c]
                a00 = jnp.where(h00 == c, jnp.take_along_axis(chunk, l00, axis=1), a00)
                a01 = jnp.where(h01 == c, jnp.take_along_axis(chunk, l01, axis=1), a01)
                a10 = jnp.where(h10 == c, jnp.take_along_axis(chunk, l10, axis=1), a10)
                a11 = jnp.where(h11 == c, jnp.take_along_axis(chunk, l11, axis=1), a11)
                return a00, a01, a10, a11

            z = jnp.zeros((Cg, LANES), jnp.float32)
            a00, a01, a10, a11 = jax.lax.fori_loop(clo, chi + 1, body, (z, z, z, z))

            w00 = (1 - ty) * (1 - tx)
            w01 = (1 - ty) * tx
            w10 = ty * (1 - tx)
            w11 = ty * tx
            v = (a00 * v00) * w00 + (a01 * v01) * w01 + (a10 * v10) * w10 + (a11 * v11) * w11
            v = v * mm
            o_ref[0, 0, g, k] = v


def _deform_gather_v(x_refs, offs, msig, G):
    T, C, H, W = x_refs.shape
    K = 9
    GK = G * K
    Cg = C // G
    HW = H * W
    Nc = (HW + LANES - 1) // LANES
    NP = Nc * LANES
    pad = NP - HW

    xf = x_refs.reshape(T, G, Cg, HW)
    xf = jnp.pad(xf, ((0, 0), (0, 0), (0, 0), (0, pad)))
    xg = xf.reshape(T, G, Cg, Nc, LANES).transpose(0, 1, 3, 2, 4)

    of = offs.reshape(T, 2 * GK, HW)
    of = jnp.pad(of, ((0, 0), (0, 0), (0, pad))).reshape(T, 2 * GK, Nc, LANES)
    of_t = of.transpose(0, 2, 1, 3)
    mf = msig.reshape(T, GK, HW)
    mf = jnp.pad(mf, ((0, 0), (0, 0), (0, pad))).reshape(T, GK, Nc, LANES)
    mf_t = mf.transpose(0, 2, 1, 3)

    posm = jnp.arange(NP, dtype=jnp.int32)
    yi_m = posm // W
    ym = yi_m.astype(jnp.float32)
    xm = (posm - yi_m * W).astype(jnp.float32)
    yx = jnp.stack([ym, xm], 0).reshape(2, Nc, LANES).transpose(1, 0, 2)

    ky = jnp.repeat(jnp.arange(3, dtype=jnp.float32) - 1.0, 3)
    kx = jnp.tile(jnp.arange(3, dtype=jnp.float32) - 1.0, 3)
    dy = of[:, 0::2]
    dx = of[:, 1::2]
    kyb = jnp.tile(ky, G)[None, :, None, None]
    kxb = jnp.tile(kx, G)[None, :, None, None]
    py = (ym.reshape(Nc, LANES)[None, None] + kyb) + dy
    px = (xm.reshape(Nc, LANES)[None, None] + kxb) + dx
    y0 = jnp.floor(py)
    x0 = jnp.floor(px)
    yc0 = jnp.clip(y0, 0, H - 1).astype(jnp.int32)
    xc0 = jnp.clip(x0, 0, W - 1).astype(jnp.int32)
    yc1 = jnp.clip(y0 + 1.0, 0, H - 1).astype(jnp.int32)
    xc1 = jnp.clip(x0 + 1.0, 0, W - 1).astype(jnp.int32)
    idx_lo = yc0 * W + xc0
    idx_hi = yc1 * W + xc1
    clo = (idx_lo >> 7).min(axis=3)
    chi = (idx_hi >> 7).max(axis=3)
    bounds = jnp.stack([clo, chi], axis=-1)
    bounds = bounds.transpose(0, 2, 1, 3).reshape(T, Nc, 1, 2 * GK).astype(jnp.int32)

    kern = functools.partial(_gather_kernel, H, W, G, Cg)
    out = pl.pallas_call(
        kern,
        grid=(T, Nc),
        in_specs=[
            pl.BlockSpec((1, 1, 1, 2 * GK), lambda t, i: (t, i, 0, 0), memory_space=pltpu.SMEM),
            pl.BlockSpec((1, 2, LANES), lambda t, i: (i, 0, 0)),
            pl.BlockSpec((1, G, Nc, Cg, LANES), lambda t, i: (t, 0, 0, 0, 0)),
            pl.BlockSpec((1, 1, 2 * GK, LANES), lambda t, i: (t, i, 0, 0)),
            pl.BlockSpec((1, 1, GK, LANES), lambda t, i: (t, i, 0, 0)),
        ],
        out_specs=pl.BlockSpec((1, 1, G, 9, Cg, LANES), lambda t, i: (t, i, 0, 0, 0, 0)),
        out_shape=jax.ShapeDtypeStruct((T, Nc, G, 9, Cg, LANES), jnp.float32),
        compiler_params=pltpu.CompilerParams(
            dimension_semantics=("parallel", "arbitrary"),
        ),
    )(bounds, yx, xg, of_t, mf_t)

    v = out.transpose(0, 2, 4, 3, 1, 5)
    v = v.reshape(T, G, Cg, 9, NP)[..., :HW].reshape(T, G, Cg, 9, H, W)
    return v


def _mdconv_pallas_batch(x_refs, offsets, masks, w, b, G):
    T, C, Hh, Ww = x_refs.shape
    K = 9
    Cg = C // G
    Cout = w.shape[0]
    offs_b = jnp.concatenate(offsets, 0)
    mm_b = jnp.concatenate(masks, 0)
    v = _deform_gather_v(x_refs, offs_b, jax.nn.sigmoid(mm_b), G)
    outs = []
    for i in range(T):
        vi = v[i][None]
        out = jnp.einsum('bgckhw,ogck->bohw', vi, w.reshape(Cout, G, Cg, K))
        outs.append(out + b[None, :, None, None])
    return outs


def _bilinear_gather_ref(xg, py, px, Hh, Ww):
    y0 = jnp.floor(py); x0 = jnp.floor(px)
    ty = py - y0; tx = px - x0
    def g(yi, xi):
        valid = ((yi >= 0) & (yi <= Hh - 1) & (xi >= 0) & (xi <= Ww - 1)).astype(xg.dtype)
        yc = jnp.clip(yi, 0, Hh - 1).astype(jnp.int32)
        xc = jnp.clip(xi, 0, Ww - 1).astype(jnp.int32)
        Bn, Gn, Kn, Hn, Wn = yc.shape
        idx = (yc * Ww + xc).reshape(Bn, Gn, 1, Kn * Hn * Wn)
        v = jnp.take_along_axis(xg, idx, axis=3).reshape(Bn, Gn, xg.shape[2], Kn, Hn, Wn)
        return v * valid[:, :, None]
    return (g(y0, x0) * ((1 - ty) * (1 - tx))[:, :, None]
            + g(y0, x0 + 1) * ((1 - ty) * tx)[:, :, None]
            + g(y0 + 1, x0) * (ty * (1 - tx))[:, :, None]
            + g(y0 + 1, x0 + 1) * (ty * tx)[:, :, None])


def _mdconv_ref(x, offset, mask, w, b, G):
    Bn, C, Hh, Ww = x.shape
    K = 9; Cg = C // G; Cout = w.shape[0]
    off = offset.reshape(Bn, G, K, 2, Hh, Ww)
    dy, dx = off[:, :, :, 0], off[:, :, :, 1]
    m = mask.reshape(Bn, G, K, Hh, Ww)
    kk = jnp.arange(3, dtype=x.dtype) - 1.0
    ky = jnp.repeat(kk, 3); kx = jnp.tile(kk, 3)
    py = jnp.arange(Hh, dtype=x.dtype)[None, None, None, :, None] + ky[None, None, :, None, None] + dy
    px = jnp.arange(Ww, dtype=x.dtype)[None, None, None, None, :] + kx[None, None, :, None, None] + dx
    xg = x.reshape(Bn, G, Cg, Hh * Ww)
    v = _bilinear_gather_ref(xg, py, px, Hh, Ww) * m[:, :, None]
    out = jnp.einsum('bgckhw,ogck->bohw', v, w.reshape(Cout, G, Cg, K))
    return out + b[None, :, None, None]


def _dyn_offsets(off_feat, pre_offset, p, G):
    o = _conv2d(off_feat, *p['offm'])
    o1, o2, mm = jnp.split(o, 3, axis=1)
    offset = jnp.concatenate([o1, o2], axis=1)
    pre = jnp.tile(pre_offset, (1, G, 1, 1, 1))
    Bn, GK, hh, ww, _ = pre.shape
    pre_r = jnp.stack([pre[..., 1], pre[..., 0]], axis=2).reshape(Bn, 2 * GK, hh, ww)
    return offset + pre_r, mm


def _mrapa(target, refs, p):
    n, _, h_in, w_in = target.shape
    t = refs.shape[0]
    tp = _spatial_pad(target)
    rb = _spatial_pad(jnp.swapaxes(refs, 0, 1).reshape(n * t, refs.shape[2], refs.shape[3], refs.shape[4]))
    hp, wp = tp.shape[-2], tp.shape[-1]
    C = p['we1'][0].shape[0]
    emb_t = _prelu(_conv2d(tp, *p['we1']), p['a1']) * (C ** -0.5)
    emb_r = _prelu(_conv2d(rb, *p['we2']), p['a2']).reshape(n, t, C, hp, wp)
    ass = _conv2d(rb, *p['wass']).reshape(n, t, 2 * C, hp, wp)
    prob = jax.nn.softmax(jnp.einsum('nchw,ntchw->nthw', emb_t, emb_r), axis=1)
    fused = jnp.einsum('nthw,ntchw->nchw', prob, ass)
    attn = _lrelu(_conv2d(jnp.concatenate([tp, fused], axis=1), *p['wsa']))
    amul = jax.nn.sigmoid(_conv2d(_lrelu(_conv2d(attn, *p['wm1'])), *p['wm2']))
    aadd = _conv2d(_lrelu(_conv2d(attn, *p['wa1'])), *p['wa2'])
    fused = fused * amul * 2 + aadd
    feat = _lrelu(_conv2d(jnp.concatenate([tp, fused], axis=1), *p['wfus']))
    return feat[:, :, :h_in, :w_in]


def _scale_stage(x, refs, pres, p, pref, use_pallas):
    dp = p['dyn_' + pref]
    offsets, masks = [], []
    for i in range(refs.shape[0]):
        off = _lrelu(_conv2d(jnp.concatenate([x, refs[i]], axis=1), *p['oc1_' + pref]))
        off = _lrelu(_conv2d(off, *p['oc2_' + pref]))
        offset, mm = _dyn_offsets(off, pres[i], dp, DEF_GROUPS)
        offsets.append(offset)
        masks.append(mm)
    if use_pallas:
        agg = _mdconv_pallas_batch(refs[:, 0], offsets, masks, dp['w'][0], dp['w'][1], DEF_GROUPS)
    else:
        agg = [_mdconv_ref(refs[i], offsets[i], jax.nn.sigmoid(masks[i]), dp['w'][0], dp['w'][1], DEF_GROUPS)
               for i in range(refs.shape[0])]
    sw = [_lrelu(a) for a in agg]
    h = _mrapa(x, jnp.stack(sw, 0), p['head_' + pref])
    return _run_body(h, p['body_' + pref]) + x


def kernel(x, pre_offset_r3, pre_offset_r2, pre_offset_r1, ref_r3, ref_r2, ref_r1, params):
    Bn, _, Hh, Ww = x.shape
    base = jax.image.resize(x, (Bn, 3, Hh * 4, Ww * 4), method='bilinear')
    feat = _run_body(_lrelu(_conv2d(x, *params['ce']['first'])), params['ce']['body'])
    p = params['dar']
    h = _scale_stage(feat, ref_r3, pre_offset_r3, p, 's', False)
    xx = _lrelu(_pixel_shuffle(_conv2d(h, *p['tail_s']), 2))
    h = _scale_stage(xx, ref_r2, pre_offset_r2, p, 'm', False)
    xx = _lrelu(_pixel_shuffle(_conv2d(h, *p['tail_m']), 2))
    h = _scale_stage(xx, ref_r1, pre_offset_r1, p, 'l', True)
    out = _conv2d(_lrelu(_conv2d(h, *p['tail_l1'])), *p['tail_l2'])
    return out + base


# speedup vs baseline: 3.0571x; 3.0571x over previous
import functools

import jax
import jax.numpy as jnp
from jax.experimental import pallas as pl
from jax.experimental.pallas import tpu as pltpu

NGF = 64
N_BLOCKS = 16
DEF_GROUPS = 8
T_REFS = 2
LANES = 128


def _conv2d(x, w, b):
    out = jax.lax.conv_general_dilated(x, w, (1, 1), 'SAME', dimension_numbers=('NCHW', 'OIHW', 'NCHW'))
    return out + b[None, :, None, None]


def _lrelu(x):
    return jnp.where(x >= 0, x, 0.1 * x)


def _prelu(x, a):
    return jnp.where(x >= 0, x, a[None, :, None, None] * x)


def _resblock(x, p):
    return x + _conv2d(jax.nn.relu(_conv2d(x, p['w1'], p['b1'])), p['w2'], p['b2'])


def _run_body(x, plist):
    for p in plist:
        x = _resblock(x, p)
    return x


def _pixel_shuffle(x, r):
    b, c, h, w = x.shape
    x = x.reshape(b, c // (r * r), r, r, h, w)
    return x.transpose(0, 1, 4, 2, 5, 3).reshape(b, c // (r * r), h * r, w * r)


def _spatial_pad(f):
    h, w = f.shape[-2:]
    ph, pw = (4 - h % 4) % 4, (4 - w % 4) % 4
    if ph or pw:
        f = jnp.pad(f, ((0, 0), (0, 0), (0, ph), (0, pw)), mode='reflect')
    return f


def _gather_kernel(H, W, G, Cg, bnd_ref, yx_ref, xg_ref, offs_ref, m_ref, o_ref):
    K = 9
    yf = yx_ref[0, pl.ds(0, 1), :]
    xf = yx_ref[0, pl.ds(1, 1), :]
    for g in range(G):
        for k in range(K):
            ch = g * K + k
            kyf = float(k // 3 - 1)
            kxf = float(k % 3 - 1)
            dy = offs_ref[0, 0, pl.ds(2 * ch, 1), :]
            dx = offs_ref[0, 0, pl.ds(2 * ch + 1, 1), :]
            mm = m_ref[0, 0, pl.ds(ch, 1), :]
            py = (yf + kyf) + dy
            px = (xf + kxf) + dx
            y0 = jnp.floor(py)
            x0 = jnp.floor(px)
            ty = py - y0
            tx = px - x0
            y1 = y0 + 1.0
            x1 = x0 + 1.0

            def cdata(yi, xi):
                valid = ((yi >= 0) & (yi <= H - 1) & (xi >= 0) & (xi <= W - 1)).astype(jnp.float32)
                yc = jnp.clip(yi, 0, H - 1).astype(jnp.int32)
                xc = jnp.clip(xi, 0, W - 1).astype(jnp.int32)
                idx = yc * W + xc
                lidx = jnp.broadcast_to(idx & (LANES - 1), (Cg, LANES))
                hi = idx >> 7
                return valid, lidx, hi

            v00, l00, h00 = cdata(y0, x0)
            v01, l01, h01 = cdata(y0, x1)
            v10, l10, h10 = cdata(y1, x0)
            v11, l11, h11 = cdata(y1, x1)

            clo = bnd_ref[0, 0, 0, 2 * ch]
            chi = bnd_ref[0, 0, 0, 2 * ch + 1]

            def body(c, carry):
                a00, a01, a10, a11 = carry
                chunk = xg_ref[0, g, c]
                a00 = jnp.where(h00 == c, jnp.take_along_axis(chunk, l00, axis=1), a00)
                a01 = jnp.where(h01 == c, jnp.take_along_axis(chunk, l01, axis=1), a01)
                a10 = jnp.where(h10 == c, jnp.take_along_axis(chunk, l10, axis=1), a10)
                a11 = jnp.where(h11 == c, jnp.take_along_axis(chunk, l11, axis=1), a11)
                return a00, a01, a10, a11

            z = jnp.zeros((Cg, LANES), jnp.float32)
            a00, a01, a10, a11 = jax.lax.fori_loop(clo, chi + 1, body, (z, z, z, z))

            w00 = (1 - ty) * (1 - tx)
            w01 = (1 - ty) * tx
            w10 = ty * (1 - tx)
            w11 = ty * tx
            v = (a00 * v00) * w00 + (a01 * v01) * w01 + (a10 * v10) * w10 + (a11 * v11) * w11
            v = v * mm
            o_ref[0, 0, g, k] = v


def _deform_gather_v(x_refs, offs, msig, G):
    T, C, H, W = x_refs.shape
    K = 9
    GK = G * K
    Cg = C // G
    HW = H * W
    Nc = (HW + LANES - 1) // LANES
    NP = Nc * LANES
    pad = NP - HW

    xf = x_refs.reshape(T, G, Cg, HW)
    xf = jnp.pad(xf, ((0, 0), (0, 0), (0, 0), (0, pad)))
    xg = xf.reshape(T, G, Cg, Nc, LANES).transpose(0, 1, 3, 2, 4)

    of = offs.reshape(T, 2 * GK, HW)
    of = jnp.pad(of, ((0, 0), (0, 0), (0, pad))).reshape(T, 2 * GK, Nc, LANES)
    of_t = of.transpose(0, 2, 1, 3)
    mf = msig.reshape(T, GK, HW)
    mf = jnp.pad(mf, ((0, 0), (0, 0), (0, pad))).reshape(T, GK, Nc, LANES)
    mf_t = mf.transpose(0, 2, 1, 3)

    posm = jnp.arange(NP, dtype=jnp.int32)
    yi_m = posm // W
    ym = yi_m.astype(jnp.float32)
    xm = (posm - yi_m * W).astype(jnp.float32)
    yx = jnp.stack([ym, xm], 0).reshape(2, Nc, LANES).transpose(1, 0, 2)

    ky = jnp.repeat(jnp.arange(3, dtype=jnp.float32) - 1.0, 3)
    kx = jnp.tile(jnp.arange(3, dtype=jnp.float32) - 1.0, 3)
    dy = of[:, 0::2]
    dx = of[:, 1::2]
    kyb = jnp.tile(ky, G)[None, :, None, None]
    kxb = jnp.tile(kx, G)[None, :, None, None]
    py = (ym.reshape(Nc, LANES)[None, None] + kyb) + dy
    px = (xm.reshape(Nc, LANES)[None, None] + kxb) + dx
    y0 = jnp.floor(py)
    x0 = jnp.floor(px)
    big = jnp.int32(2 ** 30)
    clo = None
    chi = None
    for ay in (0.0, 1.0):
        for ax in (0.0, 1.0):
            yi = y0 + ay
            xi = x0 + ax
            val = (yi >= 0) & (yi <= H - 1) & (xi >= 0) & (xi <= W - 1)
            yc = jnp.clip(yi, 0, H - 1).astype(jnp.int32)
            xc = jnp.clip(xi, 0, W - 1).astype(jnp.int32)
            idx = yc * W + xc
            lo = jnp.where(val, idx, big)
            hi = jnp.where(val, idx, -1)
            clo = lo if clo is None else jnp.minimum(clo, lo)
            chi = hi if chi is None else jnp.maximum(chi, hi)
    clo = (clo.min(axis=3) >> 7).astype(jnp.int32)
    chi = (chi.max(axis=3) >> 7).astype(jnp.int32)
    clo = jnp.minimum(clo, Nc - 1)
    chi = jnp.minimum(chi, Nc - 1)
    bounds = jnp.stack([clo, chi], axis=-1)
    bounds = bounds.transpose(0, 2, 1, 3).reshape(T, Nc, 1, 2 * GK).astype(jnp.int32)

    kern = functools.partial(_gather_kernel, H, W, G, Cg)
    out = pl.pallas_call(
        kern,
        grid=(T, Nc),
        in_specs=[
            pl.BlockSpec((1, 1, 1, 2 * GK), lambda t, i: (t, i, 0, 0), memory_space=pltpu.SMEM),
            pl.BlockSpec((1, 2, LANES), lambda t, i: (i, 0, 0)),
            pl.BlockSpec((1, G, Nc, Cg, LANES), lambda t, i: (t, 0, 0, 0, 0)),
            pl.BlockSpec((1, 1, 2 * GK, LANES), lambda t, i: (t, i, 0, 0)),
            pl.BlockSpec((1, 1, GK, LANES), lambda t, i: (t, i, 0, 0)),
        ],
        out_specs=pl.BlockSpec((1, 1, G, 9, Cg, LANES), lambda t, i: (t, i, 0, 0, 0, 0)),
        out_shape=jax.ShapeDtypeStruct((T, Nc, G, 9, Cg, LANES), jnp.float32),
        compiler_params=pltpu.CompilerParams(
            dimension_semantics=("parallel", "arbitrary"),
        ),
    )(bounds, yx, xg, of_t, mf_t)

    v = out.transpose(0, 2, 4, 3, 1, 5)
    v = v.reshape(T, G, Cg, 9, NP)[..., :HW].reshape(T, G, Cg, 9, H, W)
    return v


def _mdconv_pallas_batch(x_refs, offsets, masks, w, b, G):
    T, C, Hh, Ww = x_refs.shape
    K = 9
    Cg = C // G
    Cout = w.shape[0]
    offs_b = jnp.concatenate(offsets, 0)
    mm_b = jnp.concatenate(masks, 0)
    v = _deform_gather_v(x_refs, offs_b, jax.nn.sigmoid(mm_b), G)
    outs = []
    for i in range(T):
        vi = v[i][None]
        out = jnp.einsum('bgckhw,ogck->bohw', vi, w.reshape(Cout, G, Cg, K))
        outs.append(out + b[None, :, None, None])
    return outs


def _bilinear_gather_ref(xg, py, px, Hh, Ww):
    y0 = jnp.floor(py); x0 = jnp.floor(px)
    ty = py - y0; tx = px - x0
    def g(yi, xi):
        valid = ((yi >= 0) & (yi <= Hh - 1) & (xi >= 0) & (xi <= Ww - 1)).astype(xg.dtype)
        yc = jnp.clip(yi, 0, Hh - 1).astype(jnp.int32)
        xc = jnp.clip(xi, 0, Ww - 1).astype(jnp.int32)
        Bn, Gn, Kn, Hn, Wn = yc.shape
        idx = (yc * Ww + xc).reshape(Bn, Gn, 1, Kn * Hn * Wn)
        v = jnp.take_along_axis(xg, idx, axis=3).reshape(Bn, Gn, xg.shape[2], Kn, Hn, Wn)
        return v * valid[:, :, None]
    return (g(y0, x0) * ((1 - ty) * (1 - tx))[:, :, None]
            + g(y0, x0 + 1) * ((1 - ty) * tx)[:, :, None]
            + g(y0 + 1, x0) * (ty * (1 - tx))[:, :, None]
            + g(y0 + 1, x0 + 1) * (ty * tx)[:, :, None])


def _mdconv_ref(x, offset, mask, w, b, G):
    Bn, C, Hh, Ww = x.shape
    K = 9; Cg = C // G; Cout = w.shape[0]
    off = offset.reshape(Bn, G, K, 2, Hh, Ww)
    dy, dx = off[:, :, :, 0], off[:, :, :, 1]
    m = mask.reshape(Bn, G, K, Hh, Ww)
    kk = jnp.arange(3, dtype=x.dtype) - 1.0
    ky = jnp.repeat(kk, 3); kx = jnp.tile(kk, 3)
    py = jnp.arange(Hh, dtype=x.dtype)[None, None, None, :, None] + ky[None, None, :, None, None] + dy
    px = jnp.arange(Ww, dtype=x.dtype)[None, None, None, None, :] + kx[None, None, :, None, None] + dx
    xg = x.reshape(Bn, G, Cg, Hh * Ww)
    v = _bilinear_gather_ref(xg, py, px, Hh, Ww) * m[:, :, None]
    out = jnp.einsum('bgckhw,ogck->bohw', v, w.reshape(Cout, G, Cg, K))
    return out + b[None, :, None, None]


def _dyn_offsets(off_feat, pre_offset, p, G):
    o = _conv2d(off_feat, *p['offm'])
    o1, o2, mm = jnp.split(o, 3, axis=1)
    offset = jnp.concatenate([o1, o2], axis=1)
    pre = jnp.tile(pre_offset, (1, G, 1, 1, 1))
    Bn, GK, hh, ww, _ = pre.shape
    pre_r = jnp.stack([pre[..., 1], pre[..., 0]], axis=2).reshape(Bn, 2 * GK, hh, ww)
    return offset + pre_r, mm


def _mrapa(target, refs, p):
    n, _, h_in, w_in = target.shape
    t = refs.shape[0]
    tp = _spatial_pad(target)
    rb = _spatial_pad(jnp.swapaxes(refs, 0, 1).reshape(n * t, refs.shape[2], refs.shape[3], refs.shape[4]))
    hp, wp = tp.shape[-2], tp.shape[-1]
    C = p['we1'][0].shape[0]
    emb_t = _prelu(_conv2d(tp, *p['we1']), p['a1']) * (C ** -0.5)
    emb_r = _prelu(_conv2d(rb, *p['we2']), p['a2']).reshape(n, t, C, hp, wp)
    ass = _conv2d(rb, *p['wass']).reshape(n, t, 2 * C, hp, wp)
    prob = jax.nn.softmax(jnp.einsum('nchw,ntchw->nthw', emb_t, emb_r), axis=1)
    fused = jnp.einsum('nthw,ntchw->nchw', prob, ass)
    attn = _lrelu(_conv2d(jnp.concatenate([tp, fused], axis=1), *p['wsa']))
    amul = jax.nn.sigmoid(_conv2d(_lrelu(_conv2d(attn, *p['wm1'])), *p['wm2']))
    aadd = _conv2d(_lrelu(_conv2d(attn, *p['wa1'])), *p['wa2'])
    fused = fused * amul * 2 + aadd
    feat = _lrelu(_conv2d(jnp.concatenate([tp, fused], axis=1), *p['wfus']))
    return feat[:, :, :h_in, :w_in]


def _scale_stage(x, refs, pres, p, pref, use_pallas):
    dp = p['dyn_' + pref]
    offsets, masks = [], []
    for i in range(refs.shape[0]):
        off = _lrelu(_conv2d(jnp.concatenate([x, refs[i]], axis=1), *p['oc1_' + pref]))
        off = _lrelu(_conv2d(off, *p['oc2_' + pref]))
        offset, mm = _dyn_offsets(off, pres[i], dp, DEF_GROUPS)
        offsets.append(offset)
        masks.append(mm)
    if use_pallas:
        agg = _mdconv_pallas_batch(refs[:, 0], offsets, masks, dp['w'][0], dp['w'][1], DEF_GROUPS)
    else:
        agg = [_mdconv_ref(refs[i], offsets[i], jax.nn.sigmoid(masks[i]), dp['w'][0], dp['w'][1], DEF_GROUPS)
               for i in range(refs.shape[0])]
    sw = [_lrelu(a) for a in agg]
    h = _mrapa(x, jnp.stack(sw, 0), p['head_' + pref])
    return _run_body(h, p['body_' + pref]) + x


def kernel(x, pre_offset_r3, pre_offset_r2, pre_offset_r1, ref_r3, ref_r2, ref_r1, params):
    Bn, _, Hh, Ww = x.shape
    base = jax.image.resize(x, (Bn, 3, Hh * 4, Ww * 4), method='bilinear')
    feat = _run_body(_lrelu(_conv2d(x, *params['ce']['first'])), params['ce']['body'])
    p = params['dar']
    h = _scale_stage(feat, ref_r3, pre_offset_r3, p, 's', False)
    xx = _lrelu(_pixel_shuffle(_conv2d(h, *p['tail_s']), 2))
    h = _scale_stage(xx, ref_r2, pre_offset_r2, p, 'm', False)
    xx = _lrelu(_pixel_shuffle(_conv2d(h, *p['tail_m']), 2))
    h = _scale_stage(xx, ref_r1, pre_offset_r1, p, 'l', True)
    out = _conv2d(_lrelu(_conv2d(h, *p['tail_l1'])), *p['tail_l2'])
    return out + base
